# Initial kernel scaffold; baseline (speedup 1.0000x reference)
import jax
import jax.numpy as jnp
from jax import lax
from jax.experimental import pallas as pl
from jax.experimental.pallas import tpu as pltpu


def _row_tile(rows, max_tile):
    t = (min(rows, max_tile) // 8) * 8
    while t >= 8:
        if rows % t == 0:
            return t
        t -= 8
    return rows


def _stats_kernel(x_ref, w_ref, st_ref):
    y = lax.dot_general(x_ref[...], w_ref[...], (((1,), (1,)), ((), ())),
                        preferred_element_type=jnp.float32)
    st_ref[0] = jnp.concatenate(
        [jnp.sum(y, axis=0, keepdims=True),
         jnp.sum(y * y, axis=0, keepdims=True)], axis=0)


def _branch_stats(x2d, wb, *, max_row_tile=2048):
    R, d_in = x2d.shape
    d_out = wb.shape[0]
    TR = _row_tile(R, max_row_tile)
    nb = R // TR
    st = pl.pallas_call(
        _stats_kernel,
        grid=(nb,),
        in_specs=[pl.BlockSpec((TR, d_in), lambda i: (i, 0)),
                  pl.BlockSpec((d_out, d_in), lambda i: (0, 0))],
        out_specs=pl.BlockSpec((1, 2, d_out), lambda i: (i, 0, 0)),
        out_shape=jax.ShapeDtypeStruct((nb, 2, d_out), jnp.float32),
        compiler_params=pltpu.CompilerParams(
            dimension_semantics=("parallel",)),
    )(x2d, wb)
    return jnp.sum(st, axis=0)


def _bn_fold(stats, rows, gamma, beta, eps=1e-5):
    mean = stats[0] / rows
    var = stats[1] / rows - mean * mean
    scale = gamma * lax.rsqrt(var + eps)
    shift = beta - mean * scale
    return scale, shift


def _interp_kernel(p2_ref, p1t_ref, x1_ref, x2_ref, wu_ref, wl_ref, bn_ref,
                   out_ref, u_scr):
    TM = p2_ref.shape[1]
    N = p1t_ref.shape[2]
    bn = bn_ref[...]

    @pl.when(pl.program_id(1) == 0)
    def _():
        yu = lax.dot_general(x1_ref[0], wu_ref[...], (((1,), (1,)), ((), ())),
                             preferred_element_type=jnp.float32)
        u_scr[...] = jnp.maximum(yu * bn[0:1] + bn[1:2], 0.0).astype(jnp.bfloat16)

    yl = lax.dot_general(x2_ref[0], wl_ref[...], (((1,), (1,)), ((), ())),
                         preferred_element_type=jnp.float32)
    lat = jnp.maximum(yl * bn[2:3] + bn[3:4], 0.0)

    p2t = p2_ref[0]
    p1t = p1t_ref[0]
    d2 = jnp.zeros((TM, N), jnp.float32)
    for c in range(3):
        diff = p2t[:, c:c + 1] - p1t[c:c + 1, :]
        d2 = d2 + diff * diff

    w = jnp.zeros((TM, N), jnp.bfloat16)
    rsum = jnp.zeros((TM, 1), jnp.float32)
    for _ in range(3):
        mn = jnp.min(d2, axis=1, keepdims=True)
        onehot = d2 == mn
        recip = 1.0 / (jnp.sqrt(jnp.maximum(mn, 0.0)) + 1e-8)
        w = jnp.where(onehot, recip.astype(jnp.bfloat16), w)
        rsum = rsum + recip
        d2 = jnp.where(onehot, jnp.float32(1e30), d2)

    interp = jnp.dot(w, u_scr[...], preferred_element_type=jnp.float32)
    out_ref[0] = interp * (1.0 / rsum) + lat


def kernel(x1, p1, x2, p2, Wu, gu, bu, Wl, gl, bl):
    B, N, d_in = x1.shape
    M = x2.shape[1]
    d_out = Wu.shape[0]

    xc1 = x1.astype(jnp.bfloat16)
    xc2 = x2.astype(jnp.bfloat16)
    wub = Wu.astype(jnp.bfloat16)
    wlb = Wl.astype(jnp.bfloat16)

    u_stats = _branch_stats(xc1.reshape(B * N, d_in), wub)
    l_stats = _branch_stats(xc2.reshape(B * M, d_out), wlb)
    su, tu = _bn_fold(u_stats, B * N, gu, bu)
    sl, tl = _bn_fold(l_stats, B * M, gl, bl)
    bn_slab = jnp.stack([su, tu, sl, tl], axis=0)

    p1t = jnp.transpose(p1, (0, 2, 1))
    TM = _row_tile(M, 256)
    grid = (B, M // TM)
    y = pl.pallas_call(
        _interp_kernel,
        grid=grid,
        in_specs=[pl.BlockSpec((1, TM, 3), lambda b, m: (b, m, 0)),
                  pl.BlockSpec((1, 3, N), lambda b, m: (b, 0, 0)),
                  pl.BlockSpec((1, N, d_in), lambda b, m: (b, 0, 0)),
                  pl.BlockSpec((1, TM, d_out), lambda b, m: (b, m, 0)),
                  pl.BlockSpec((d_out, d_in), lambda b, m: (0, 0)),
                  pl.BlockSpec((d_out, d_out), lambda b, m: (0, 0)),
                  pl.BlockSpec((4, d_out), lambda b, m: (0, 0))],
        out_specs=pl.BlockSpec((1, TM, d_out), lambda b, m: (b, m, 0)),
        out_shape=jax.ShapeDtypeStruct((B, M, d_out), jnp.float32),
        scratch_shapes=[pltpu.VMEM((N, d_out), jnp.bfloat16)],
        compiler_params=pltpu.CompilerParams(
            dimension_semantics=("parallel", "arbitrary")),
    )(p2, p1t, xc1, xc2, wub, wlb, bn_slab)
    return y, p2


# speedup vs baseline: 1.3734x; 1.3734x over previous
import jax
import jax.numpy as jnp
from jax import lax
from jax.experimental import pallas as pl
from jax.experimental.pallas import tpu as pltpu


def _row_tile(rows, max_tile):
    t = (min(rows, max_tile) // 8) * 8
    while t >= 8:
        if rows % t == 0:
            return t
        t -= 8
    return rows


def _stats_kernel(x_ref, w_ref, st_ref):
    y = lax.dot_general(x_ref[...], w_ref[...], (((1,), (1,)), ((), ())),
                        preferred_element_type=jnp.float32)
    st_ref[0] = jnp.concatenate(
        [jnp.sum(y, axis=0, keepdims=True),
         jnp.sum(y * y, axis=0, keepdims=True)], axis=0)


def _branch_stats(x2d, wb, *, max_row_tile=2048):
    R, d_in = x2d.shape
    d_out = wb.shape[0]
    TR = _row_tile(R, max_row_tile)
    nb = R // TR
    st = pl.pallas_call(
        _stats_kernel,
        grid=(nb,),
        in_specs=[pl.BlockSpec((TR, d_in), lambda i: (i, 0)),
                  pl.BlockSpec((d_out, d_in), lambda i: (0, 0))],
        out_specs=pl.BlockSpec((1, 2, d_out), lambda i: (i, 0, 0)),
        out_shape=jax.ShapeDtypeStruct((nb, 2, d_out), jnp.float32),
        compiler_params=pltpu.CompilerParams(
            dimension_semantics=("parallel",)),
    )(x2d, wb)
    return jnp.sum(st, axis=0)


def _bn_fold(stats, rows, gamma, beta, eps=1e-5):
    mean = stats[0] / rows
    var = stats[1] / rows - mean * mean
    scale = gamma * lax.rsqrt(var + eps)
    shift = beta - mean * scale
    return scale, shift


def _interp_kernel(p2_ref, p1t_ref, x1_ref, x2_ref, wu_ref, wl_ref, bn_ref,
                   out_ref, u_scr):
    TM = p2_ref.shape[1]
    N = p1t_ref.shape[2]
    bn = bn_ref[...]

    @pl.when(pl.program_id(1) == 0)
    def _():
        yu = lax.dot_general(x1_ref[0], wu_ref[...], (((1,), (1,)), ((), ())),
                             preferred_element_type=jnp.float32)
        u_scr[...] = jnp.maximum(yu * bn[0:1] + bn[1:2], 0.0).astype(jnp.bfloat16)

    yl = lax.dot_general(x2_ref[0], wl_ref[...], (((1,), (1,)), ((), ())),
                         preferred_element_type=jnp.float32)
    lat = jnp.maximum(yl * bn[2:3] + bn[3:4], 0.0)

    p2t = p2_ref[0]
    p1t = p1t_ref[0]
    d2 = jnp.zeros((TM, N), jnp.float32)
    for c in range(3):
        diff = p2t[:, c:c + 1] - p1t[c:c + 1, :]
        d2 = d2 + diff * diff

    w = jnp.zeros((TM, N), jnp.float32)
    rsum = jnp.zeros((TM, 1), jnp.float32)
    for _ in range(3):
        mn = jnp.min(d2, axis=1, keepdims=True)
        onehot = d2 == mn
        recip = 1.0 / (jnp.sqrt(jnp.maximum(mn, 0.0)) + 1e-8)
        w = jnp.where(onehot, recip, w)
        rsum = rsum + recip
        d2 = jnp.where(onehot, jnp.float32(1e30), d2)

    interp = jnp.dot(w.astype(jnp.bfloat16), u_scr[...],
                     preferred_element_type=jnp.float32)
    out_ref[0] = interp * (1.0 / rsum) + lat


def kernel(x1, p1, x2, p2, Wu, gu, bu, Wl, gl, bl):
    B, N, d_in = x1.shape
    M = x2.shape[1]
    d_out = Wu.shape[0]

    xc1 = x1.astype(jnp.bfloat16)
    xc2 = x2.astype(jnp.bfloat16)
    wub = Wu.astype(jnp.bfloat16)
    wlb = Wl.astype(jnp.bfloat16)

    u_stats = _branch_stats(xc1.reshape(B * N, d_in), wub)
    l_stats = _branch_stats(xc2.reshape(B * M, d_out), wlb)
    su, tu = _bn_fold(u_stats, B * N, gu, bu)
    sl, tl = _bn_fold(l_stats, B * M, gl, bl)
    bn_slab = jnp.stack([su, tu, sl, tl], axis=0)

    p1t = jnp.transpose(p1, (0, 2, 1))
    TM = _row_tile(M, 256)
    grid = (B, M // TM)
    y = pl.pallas_call(
        _interp_kernel,
        grid=grid,
        in_specs=[pl.BlockSpec((1, TM, 3), lambda b, m: (b, m, 0)),
                  pl.BlockSpec((1, 3, N), lambda b, m: (b, 0, 0)),
                  pl.BlockSpec((1, N, d_in), lambda b, m: (b, 0, 0)),
                  pl.BlockSpec((1, TM, d_out), lambda b, m: (b, m, 0)),
                  pl.BlockSpec((d_out, d_in), lambda b, m: (0, 0)),
                  pl.BlockSpec((d_out, d_out), lambda b, m: (0, 0)),
                  pl.BlockSpec((4, d_out), lambda b, m: (0, 0))],
        out_specs=pl.BlockSpec((1, TM, d_out), lambda b, m: (b, m, 0)),
        out_shape=jax.ShapeDtypeStruct((B, M, d_out), jnp.float32),
        scratch_shapes=[pltpu.VMEM((N, d_out), jnp.bfloat16)],
        compiler_params=pltpu.CompilerParams(
            dimension_semantics=("parallel", "arbitrary")),
    )(p2, p1t, xc1, xc2, wub, wlb, bn_slab)
    return y, p2
